# baseline (speedup 1.0000x reference)
import functools

import jax
import jax.numpy as jnp
from jax import lax
from jax.experimental import pallas as pl
from jax.experimental.pallas import tpu as pltpu
from jax.experimental.pallas import tpu_sc as plsc

HOT_ROWS = 1024
STEMS_D = 64
SFX_D = 32
OUT_D = STEMS_D + SFX_D
LANES = 16
NUM_CORES = 2
NUM_SUBCORES = 16
NUM_WORKERS = NUM_CORES * NUM_SUBCORES
BT = 128
PITCH = BT + 1


def _body(inp_hbm, stems_hbm, sfx_hbm, out_hbm, stems_v, sfx_v,
          idx_a, idx_b, out_a, out_b, sem_ia, sem_ib, sem_oa, sem_ob):
    seq = inp_hbm.shape[0]
    nbt = inp_hbm.shape[1]
    num_units = seq * nbt
    units_per_w = num_units // NUM_WORKERS
    wid = lax.axis_index("s") * NUM_CORES + lax.axis_index("c")
    u0 = wid * units_per_w
    pairs = units_per_w // 2

    pltpu.sync_copy(stems_hbm, stems_v)
    pltpu.sync_copy(sfx_hbm, sfx_v)

    lanes = lax.iota(jnp.int32, 16)

    def unit_addr(u):
        return u // nbt, u % nbt

    def start_idx(u, ibuf, isem):
        l, bt = unit_addr(u)
        pltpu.async_copy(inp_hbm.at[l, bt], ibuf, isem)

    def wait_idx(u, ibuf, isem):
        l, bt = unit_addr(u)
        pltpu.make_async_copy(inp_hbm.at[l, bt], ibuf, isem).wait()

    def start_out(u, obuf, osem):
        l, bt = unit_addr(u)
        pltpu.async_copy(obuf.at[:, :, pl.ds(0, BT)],
                         out_hbm.at[l, :, bt], osem)

    def wait_out(u, obuf, osem):
        l, bt = unit_addr(u)
        pltpu.make_async_copy(obuf.at[:, :, pl.ds(0, BT)],
                              out_hbm.at[l, :, bt], osem).wait()

    def compute(ibuf, obuf):
        @plsc.parallel_loop(0, BT // LANES, unroll=4)
        def group_body(g):
            i0 = ibuf[0, pl.ds(g * LANES, LANES)]
            i1 = ibuf[1, pl.ds(g * LANES, LANES)]
            slab8 = lanes // 8
            row8 = lanes % 8
            for tok in range(LANES):
                t = g * LANES + tok
                sel = jnp.full((LANES,), tok, jnp.int32)
                r0 = jnp.take_along_axis(i0, sel, axis=0,
                                         mode="promise_in_bounds")
                r1 = jnp.take_along_axis(i1, sel, axis=0,
                                         mode="promise_in_bounds")
                col = jnp.full((LANES,), t, jnp.int32)
                for j in range(OUT_D // LANES):
                    d0 = j * LANES
                    if d0 < STEMS_D:
                        v = plsc.load_gather(stems_v, [r0, d0 + lanes])
                    else:
                        v = plsc.load_gather(sfx_v, [r1, d0 - STEMS_D + lanes])
                    plsc.store_scatter(obuf, [2 * j + slab8, row8, col], v)

    bufs = ((idx_a, sem_ia, out_a, sem_oa), (idx_b, sem_ib, out_b, sem_ob))

    start_idx(u0 + 0, idx_a, sem_ia)
    start_idx(u0 + 1, idx_b, sem_ib)
    for par, (ibuf, isem, obuf, osem) in enumerate(bufs):
        u = u0 + par
        wait_idx(u, ibuf, isem)
        compute(ibuf, obuf)
        start_idx(u + 2, ibuf, isem)
        start_out(u, obuf, osem)

    def pair_body(ci, carry):
        for par, (ibuf, isem, obuf, osem) in enumerate(bufs):
            u = u0 + 2 * ci + par
            wait_idx(u, ibuf, isem)
            wait_out(u - 2, obuf, osem)
            compute(ibuf, obuf)

            @pl.when(ci < pairs - 1)
            def _():
                start_idx(u + 2, ibuf, isem)

            start_out(u, obuf, osem)
        return carry

    lax.fori_loop(1, pairs, pair_body, 0)
    wait_out(u0 + units_per_w - 2, out_a, sem_oa)
    wait_out(u0 + units_per_w - 1, out_b, sem_ob)


def kernel(input, training, stems_table, sfx_table):
    del training
    b, s, _ = input.shape
    nbt = b // BT

    inp_p = input.reshape(nbt, BT, s, 2).transpose(2, 0, 3, 1)
    stems_hot = lax.slice(stems_table, (0, 0), (HOT_ROWS, STEMS_D))

    mesh = plsc.VectorSubcoreMesh(core_axis_name="c", subcore_axis_name="s")
    k = pl.kernel(
        _body,
        out_type=jax.ShapeDtypeStruct((s, OUT_D // 8, nbt, 8, BT), jnp.float32),
        mesh=mesh,
        compiler_params=pltpu.CompilerParams(
            needs_layout_passes=False, use_tc_tiling_on_sc=False),
        scratch_types=[
            pltpu.VMEM((HOT_ROWS, STEMS_D), jnp.float32),
            pltpu.VMEM(sfx_table.shape, jnp.float32),
            pltpu.VMEM((2, BT), jnp.int32),
            pltpu.VMEM((2, BT), jnp.int32),
            pltpu.VMEM((OUT_D // 8, 8, PITCH), jnp.float32),
            pltpu.VMEM((OUT_D // 8, 8, PITCH), jnp.float32),
            pltpu.SemaphoreType.DMA,
            pltpu.SemaphoreType.DMA,
            pltpu.SemaphoreType.DMA,
            pltpu.SemaphoreType.DMA,
        ],
    )
    out5 = k(inp_p, stems_hot, sfx_table)
    return out5.transpose(2, 4, 0, 1, 3).reshape(b, s, OUT_D)


# speedup vs baseline: 1.2904x; 1.2904x over previous
import functools

import numpy as np

import jax
import jax.numpy as jnp
from jax import lax
from jax.experimental import pallas as pl
from jax.experimental.pallas import tpu as pltpu
from jax.experimental.pallas import tpu_sc as plsc

HOT_ROWS = 1024
STEMS_D = 64
SFX_D = 32
OUT_D = STEMS_D + SFX_D
LANES = 16
NUM_CORES = 2
NUM_SUBCORES = 16
NUM_WORKERS = NUM_CORES * NUM_SUBCORES
BT = 128
PITCH = BT + 1


def _body(inp_hbm, stems_hbm, sfx_hbm, out_hbm, stems_v, sfx_v,
          idx_a, idx_b, out_a, out_b, sem_ia, sem_ib, sem_oa, sem_ob):
    seq = inp_hbm.shape[0]
    nbt = inp_hbm.shape[1]
    num_units = seq * nbt
    units_per_w = num_units // NUM_WORKERS
    wid = lax.axis_index("s") * NUM_CORES + lax.axis_index("c")
    u0 = wid * units_per_w
    pairs = units_per_w // 2

    pltpu.sync_copy(stems_hbm, stems_v)
    pltpu.sync_copy(sfx_hbm, sfx_v)

    def unit_addr(u):
        return u // nbt, u % nbt

    def start_idx(u, ibuf, isem):
        l, bt = unit_addr(u)
        pltpu.async_copy(inp_hbm.at[l, bt], ibuf, isem)

    def wait_idx(u, ibuf, isem):
        l, bt = unit_addr(u)
        pltpu.make_async_copy(inp_hbm.at[l, bt], ibuf, isem).wait()

    def start_out(u, obuf, osem):
        l, bt = unit_addr(u)
        pltpu.async_copy(obuf.at[:, :, pl.ds(0, BT)],
                         out_hbm.at[l, :, bt], osem)

    def wait_out(u, obuf, osem):
        l, bt = unit_addr(u)
        pltpu.make_async_copy(obuf.at[:, :, pl.ds(0, BT)],
                              out_hbm.at[l, :, bt], osem).wait()

    def compute(ibuf, obuf):

        @plsc.parallel_loop(0, BT // LANES, unroll=2)
        def group_body(g):
            lanes = lax.iota(jnp.int32, LANES)
            slab8 = lanes // 8
            row8 = lanes % 8
            i0 = ibuf[0, pl.ds(g * LANES, LANES)]
            i1 = ibuf[1, pl.ds(g * LANES, LANES)]
            for tok in range(LANES):
                t = g * LANES + tok
                sel = jnp.full((LANES,), tok, jnp.int32)
                r0 = jnp.take_along_axis(i0, sel, axis=0,
                                         mode="promise_in_bounds")
                r1 = jnp.take_along_axis(i1, sel, axis=0,
                                         mode="promise_in_bounds")
                col = jnp.full((LANES,), t, jnp.int32)
                for j in range(OUT_D // LANES):
                    d0 = j * LANES
                    if d0 < STEMS_D:
                        v = plsc.load_gather(stems_v, [r0, d0 + lanes])
                    else:
                        v = plsc.load_gather(sfx_v, [r1, d0 - STEMS_D + lanes])
                    plsc.store_scatter(
                        obuf, [2 * j + slab8, row8, col], v)

    bufs = ((idx_a, sem_ia, out_a, sem_oa), (idx_b, sem_ib, out_b, sem_ob))

    start_idx(u0 + 0, idx_a, sem_ia)
    start_idx(u0 + 1, idx_b, sem_ib)
    for par, (ibuf, isem, obuf, osem) in enumerate(bufs):
        u = u0 + par
        wait_idx(u, ibuf, isem)
        compute(ibuf, obuf)
        start_idx(u + 2, ibuf, isem)
        start_out(u, obuf, osem)

    def pair_body(ci, carry):
        for par, (ibuf, isem, obuf, osem) in enumerate(bufs):
            u = u0 + 2 * ci + par
            wait_idx(u, ibuf, isem)
            wait_out(u - 2, obuf, osem)
            compute(ibuf, obuf)

            @pl.when(ci < pairs - 1)
            def _():
                start_idx(u + 2, ibuf, isem)

            start_out(u, obuf, osem)
        return carry

    lax.fori_loop(1, pairs, pair_body, 0)
    wait_out(u0 + units_per_w - 2, out_a, sem_oa)
    wait_out(u0 + units_per_w - 1, out_b, sem_ob)


def kernel(input, training, stems_table, sfx_table):
    del training
    b, s, _ = input.shape
    nbt = b // BT

    inp_p = input.reshape(nbt, BT, s, 2).transpose(2, 0, 3, 1)
    stems_hot = lax.slice(stems_table, (0, 0), (HOT_ROWS, STEMS_D))

    mesh = plsc.VectorSubcoreMesh(core_axis_name="c", subcore_axis_name="s")
    k = pl.kernel(
        _body,
        out_type=jax.ShapeDtypeStruct((s, OUT_D // 8, nbt, 8, BT), jnp.float32),
        mesh=mesh,
        compiler_params=pltpu.CompilerParams(
            needs_layout_passes=False, use_tc_tiling_on_sc=False),
        scratch_types=[
            pltpu.VMEM((HOT_ROWS, STEMS_D), jnp.float32),
            pltpu.VMEM(sfx_table.shape, jnp.float32),
            pltpu.VMEM((2, BT), jnp.int32),
            pltpu.VMEM((2, BT), jnp.int32),
            pltpu.VMEM((OUT_D // 8, 8, PITCH), jnp.float32),
            pltpu.VMEM((OUT_D // 8, 8, PITCH), jnp.float32),
            pltpu.SemaphoreType.DMA,
            pltpu.SemaphoreType.DMA,
            pltpu.SemaphoreType.DMA,
            pltpu.SemaphoreType.DMA,
        ],
    )
    out5 = k(inp_p, stems_hot, sfx_table)
    return out5.transpose(2, 4, 0, 1, 3).reshape(b, s, OUT_D)


# speedup vs baseline: 1.2924x; 1.0015x over previous
import jax
import jax.numpy as jnp
from jax import lax
from jax.experimental import pallas as pl
from jax.experimental.pallas import tpu as pltpu
from jax.experimental.pallas import tpu_sc as plsc

HOT_ROWS = 1024
STEMS_D = 64
SFX_D = 32
OUT_D = STEMS_D + SFX_D
LANES = 16
NUM_CORES = 2
NUM_SUBCORES = 16
NUM_WORKERS = NUM_CORES * NUM_SUBCORES
BT = 128
PITCH = BT + 1


def _body(inp_hbm, stems_hbm, sfx_hbm, out_hbm, stems_v, sfx_v,
          idx_a, idx_b, out_a, out_b, sem_ia, sem_ib, sem_oa, sem_ob):
    seq = inp_hbm.shape[0]
    nbt = inp_hbm.shape[1]
    num_units = seq * nbt
    units_per_w = num_units // NUM_WORKERS
    wid = lax.axis_index("s") * NUM_CORES + lax.axis_index("c")
    u0 = wid * units_per_w
    pairs = units_per_w // 2

    pltpu.sync_copy(stems_hbm, stems_v)
    pltpu.sync_copy(sfx_hbm, sfx_v)

    def unit_addr(u):
        return u // nbt, u % nbt

    def start_idx(u, ibuf, isem):
        l, bt = unit_addr(u)
        pltpu.async_copy(inp_hbm.at[l, bt], ibuf, isem)

    def wait_idx(u, ibuf, isem):
        l, bt = unit_addr(u)
        pltpu.make_async_copy(inp_hbm.at[l, bt], ibuf, isem).wait()

    def start_out(u, obuf, osem):
        l, bt = unit_addr(u)
        pltpu.async_copy(obuf.at[:, :, pl.ds(0, BT)],
                         out_hbm.at[l, :, bt], osem)

    def wait_out(u, obuf, osem):
        l, bt = unit_addr(u)
        pltpu.make_async_copy(obuf.at[:, :, pl.ds(0, BT)],
                              out_hbm.at[l, :, bt], osem).wait()

    def compute(ibuf, obuf):

        @plsc.parallel_loop(0, BT // LANES, unroll=2)
        def group_body(g):
            lanes = lax.iota(jnp.int32, LANES)
            slab8 = lanes // 8
            row8 = lanes % 8
            i0 = ibuf[0, pl.ds(g * LANES, LANES)]
            i1 = ibuf[1, pl.ds(g * LANES, LANES)]
            for tok in range(LANES):
                t = g * LANES + tok
                sel = jnp.full((LANES,), tok, jnp.int32)
                r0 = jnp.take_along_axis(i0, sel, axis=0,
                                         mode="promise_in_bounds")
                r1 = jnp.take_along_axis(i1, sel, axis=0,
                                         mode="promise_in_bounds")
                col = jnp.full((LANES,), t, jnp.int32)
                for j in range(OUT_D // LANES):
                    d0 = j * LANES
                    if d0 < STEMS_D:
                        v = plsc.load_gather(stems_v, [r0, d0 + lanes])
                    else:
                        v = plsc.load_gather(sfx_v, [r1, d0 - STEMS_D + lanes])
                    plsc.store_scatter(
                        obuf, [2 * j + slab8, row8, col], v)

    bufs = ((idx_a, sem_ia, out_a, sem_oa), (idx_b, sem_ib, out_b, sem_ob))

    start_idx(u0 + 0, idx_a, sem_ia)
    start_idx(u0 + 1, idx_b, sem_ib)
    for par, (ibuf, isem, obuf, osem) in enumerate(bufs):
        u = u0 + par
        wait_idx(u, ibuf, isem)
        compute(ibuf, obuf)
        start_idx(u + 2, ibuf, isem)
        start_out(u, obuf, osem)

    def pair_body(ci, carry):
        for par, (ibuf, isem, obuf, osem) in enumerate(bufs):
            u = u0 + 2 * ci + par
            wait_idx(u, ibuf, isem)
            wait_out(u - 2, obuf, osem)
            compute(ibuf, obuf)

            @pl.when(ci < pairs - 1)
            def _():
                start_idx(u + 2, ibuf, isem)

            start_out(u, obuf, osem)
        return carry

    lax.fori_loop(1, pairs, pair_body, 0)
    wait_out(u0 + units_per_w - 2, out_a, sem_oa)
    wait_out(u0 + units_per_w - 1, out_b, sem_ob)


def kernel(input, training, stems_table, sfx_table):
    del training
    b, s, _ = input.shape
    nbt = b // BT

    inp_p = input.reshape(nbt, BT, s, 2).transpose(2, 0, 3, 1)
    stems_hot = lax.slice(stems_table, (0, 0), (HOT_ROWS, STEMS_D))

    mesh = plsc.VectorSubcoreMesh(core_axis_name="c", subcore_axis_name="s")
    k = pl.kernel(
        _body,
        out_type=jax.ShapeDtypeStruct((s, OUT_D // 8, nbt, 8, BT), jnp.float32),
        mesh=mesh,
        compiler_params=pltpu.CompilerParams(
            needs_layout_passes=False, use_tc_tiling_on_sc=False),
        scratch_types=[
            pltpu.VMEM((HOT_ROWS, STEMS_D), jnp.float32),
            pltpu.VMEM(sfx_table.shape, jnp.float32),
            pltpu.VMEM((2, BT), jnp.int32),
            pltpu.VMEM((2, BT), jnp.int32),
            pltpu.VMEM((OUT_D // 8, 8, PITCH), jnp.float32),
            pltpu.VMEM((OUT_D // 8, 8, PITCH), jnp.float32),
            pltpu.SemaphoreType.DMA,
            pltpu.SemaphoreType.DMA,
            pltpu.SemaphoreType.DMA,
            pltpu.SemaphoreType.DMA,
        ],
    )
    out5 = k(inp_p, stems_hot, sfx_table)
    return out5.transpose(2, 4, 0, 1, 3).reshape(b, s, OUT_D)
